# baseline (speedup 1.0000x reference)
import functools

import jax
import jax.numpy as jnp
from jax import lax
from jax.experimental import pallas as pl
from jax.experimental.pallas import tpu as pltpu
from jax.experimental.pallas import tpu_sc as plsc

B = 4096
L = 200
HID = 64

NC = 2
NS = 16
NW = NC * NS
BB = B // NW
NBUF = 4
NT = L // NBUF


@functools.partial(
    pl.kernel,
    mesh=plsc.VectorSubcoreMesh(core_axis_name="c", subcore_axis_name="s"),
    compiler_params=pltpu.CompilerParams(
        use_tc_tiling_on_sc=False, needs_layout_passes=False),
    out_type=jax.ShapeDtypeStruct((L, HID // 8, NW, 8 * 128), jnp.float32),
    scratch_types=[
        pltpu.VMEM((L, HID), jnp.float32),
        pltpu.VMEM((BB,), jnp.int32),
        pltpu.VMEM((BB,), jnp.int32),
        pltpu.VMEM((BB,), jnp.int32),
        pltpu.VMEM((BB,), jnp.int32),
        pltpu.VMEM((BB, HID), jnp.float32),
        pltpu.VMEM((BB, HID), jnp.float32),
        pltpu.VMEM((BB, HID), jnp.float32),
        pltpu.VMEM((BB, HID), jnp.float32),
        pltpu.VMEM((HID // 8, 8 * 128), jnp.float32),
        pltpu.VMEM((HID // 8, 8 * 128), jnp.float32),
        pltpu.VMEM((HID // 8, 8 * 128), jnp.float32),
        pltpu.VMEM((HID // 8, 8 * 128), jnp.float32),
        pltpu.SemaphoreType.DMA,
        pltpu.SemaphoreType.DMA,
        pltpu.SemaphoreType.DMA,
        pltpu.SemaphoreType.DMA,
        pltpu.SemaphoreType.DMA,
        pltpu.SemaphoreType.DMA,
        pltpu.SemaphoreType.DMA,
        pltpu.SemaphoreType.DMA,
        pltpu.SemaphoreType.DMA,
        pltpu.SemaphoreType.DMA,
        pltpu.SemaphoreType.DMA,
        pltpu.SemaphoreType.DMA,
    ],
)
def _emb_kernel(xT_hbm, tok_hbm, pos_hbm, out_hbm,
                pos_v,
                ib0, ib1, ib2, ib3,
                gb0, gb1, gb2, gb3,
                sb0, sb1, sb2, sb3,
                si0, si1, si2, si3,
                sg0, sg1, sg2, sg3,
                ss0, ss1, ss2, ss3):
    ibs = (ib0, ib1, ib2, ib3)
    gbs = (gb0, gb1, gb2, gb3)
    sbs = (sb0, sb1, sb2, sb3)
    sis = (si0, si1, si2, si3)
    sgs = (sg0, sg1, sg2, sg3)
    sss = (ss0, ss1, ss2, ss3)

    wid = lax.axis_index("s") * NC + lax.axis_index("c")
    col0 = wid * BB

    pltpu.sync_copy(pos_hbm.at[pl.ds(0, L)], pos_v)

    iota = lax.iota(jnp.int32, 16)
    rows = [iota + (b0 * 16) for b0 in range(8)]

    def issue_idx(g, k):
        pltpu.async_copy(xT_hbm.at[g, pl.ds(col0, BB)], ibs[k], sis[k])

    def wait_idx(k):
        pltpu.make_async_copy(
            xT_hbm.at[0, pl.ds(0, BB)], ibs[k], sis[k]).wait()

    def issue_gather(k):
        pltpu.async_copy(tok_hbm.at[ibs[k]], gbs[k], sgs[k])

    def wait_gather(k):
        pltpu.make_async_copy(
            tok_hbm.at[pl.ds(0, BB)], gbs[k], sgs[k]).wait()

    def issue_store(g, k):
        pltpu.async_copy(sbs[k], out_hbm.at[g, :, wid], sss[k])

    def wait_store(k):
        pltpu.make_async_copy(sbs[k], out_hbm.at[0, :, 0], sss[k]).wait()

    def transpose_add(g, k):
        gbuf = gbs[k]
        sbuf = sbs[k]
        lvec = jnp.full((16,), g, dtype=jnp.int32)

        def h_body(h, carry):
            colh = jnp.full((16,), h, dtype=jnp.int32)
            pv = plsc.load_gather(pos_v, [lvec, colh])
            th = h >> 3
            off = (h & 7) * 128
            for b0 in range(8):
                v = plsc.load_gather(gbuf, [rows[b0], colh])
                sbuf[th, pl.ds(off + b0 * 16, 16)] = v + pv
            return carry

        lax.fori_loop(0, HID, h_body, 0)

    issue_idx(0, 0)
    issue_idx(1, 1)
    wait_idx(0)
    issue_gather(0)

    def outer(t, carry):
        for b in range(NBUF):
            g = t * NBUF + b
            k1 = (b + 1) % NBUF
            k2 = (b + 2) % NBUF

            def stage1():
                wait_idx(k1)
                issue_gather(k1)

            if b == 3:
                @pl.when(t < NT - 1)
                def _():
                    stage1()
            else:
                stage1()

            def stage2():
                issue_idx(g + 2, k2)

            if b >= 2:
                @pl.when(t < NT - 1)
                def _():
                    stage2()
            else:
                stage2()

            wait_gather(b)

            @pl.when(t > 0)
            def _():
                wait_store(b)

            transpose_add(g, b)
            issue_store(g, b)
        return carry

    lax.fori_loop(0, NT, outer, 0)

    for b in range(NBUF):
        wait_store(b)


def kernel(x, token_table, pos_table):
    xT = jnp.transpose(x).astype(jnp.int32)
    o = _emb_kernel(xT, token_table, pos_table)
    o = o.reshape(L, HID // 8, NW, 8, 128)
    return o.transpose(2, 4, 0, 1, 3).reshape(B, L, HID)


# speedup vs baseline: 1.1258x; 1.1258x over previous
import functools

import jax
import jax.numpy as jnp
from jax import lax
from jax.experimental import pallas as pl
from jax.experimental.pallas import tpu as pltpu
from jax.experimental.pallas import tpu_sc as plsc

B = 4096
L = 200
HID = 64

NC = 2
NS = 16
NW = NC * NS
BB = B // NW
NBUF = 4
NT = L // NBUF


@functools.partial(
    pl.kernel,
    mesh=plsc.VectorSubcoreMesh(core_axis_name="c", subcore_axis_name="s"),
    compiler_params=pltpu.CompilerParams(
        use_tc_tiling_on_sc=False, needs_layout_passes=False),
    out_type=jax.ShapeDtypeStruct((L, HID // 8, NW, 8 * 128), jnp.float32),
    scratch_types=[
        pltpu.VMEM((L, HID), jnp.float32),
        pltpu.VMEM((BB,), jnp.int32),
        pltpu.VMEM((BB,), jnp.int32),
        pltpu.VMEM((BB,), jnp.int32),
        pltpu.VMEM((BB,), jnp.int32),
        pltpu.VMEM((BB, HID), jnp.float32),
        pltpu.VMEM((BB, HID), jnp.float32),
        pltpu.VMEM((BB, HID), jnp.float32),
        pltpu.VMEM((BB, HID), jnp.float32),
        pltpu.VMEM((HID // 8, 8 * 128), jnp.float32),
        pltpu.VMEM((HID // 8, 8 * 128), jnp.float32),
        pltpu.VMEM((HID // 8, 8 * 128), jnp.float32),
        pltpu.VMEM((HID // 8, 8 * 128), jnp.float32),
        pltpu.SemaphoreType.DMA,
        pltpu.SemaphoreType.DMA,
        pltpu.SemaphoreType.DMA,
        pltpu.SemaphoreType.DMA,
        pltpu.SemaphoreType.DMA,
        pltpu.SemaphoreType.DMA,
        pltpu.SemaphoreType.DMA,
        pltpu.SemaphoreType.DMA,
        pltpu.SemaphoreType.DMA,
        pltpu.SemaphoreType.DMA,
        pltpu.SemaphoreType.DMA,
        pltpu.SemaphoreType.DMA,
    ],
)
def _emb_kernel(xT_hbm, tok_hbm, pos_hbm, out_hbm,
                pos_v,
                ib0, ib1, ib2, ib3,
                gb0, gb1, gb2, gb3,
                sb0, sb1, sb2, sb3,
                si0, si1, si2, si3,
                sg0, sg1, sg2, sg3,
                ss0, ss1, ss2, ss3):
    ibs = (ib0, ib1, ib2, ib3)
    gbs = (gb0, gb1, gb2, gb3)
    sbs = (sb0, sb1, sb2, sb3)
    sis = (si0, si1, si2, si3)
    sgs = (sg0, sg1, sg2, sg3)
    sss = (ss0, ss1, ss2, ss3)

    wid = lax.axis_index("s") * NC + lax.axis_index("c")
    col0 = wid * BB

    pltpu.sync_copy(pos_hbm.at[pl.ds(0, L)], pos_v)

    iota = lax.iota(jnp.int32, 16)
    th_q = [(iota + 16 * q) >> 3 for q in range(4)]
    cb_q = [((iota + 16 * q) & 7) * 128 for q in range(4)]

    def issue_idx(g, k):
        pltpu.async_copy(xT_hbm.at[g, pl.ds(col0, BB)], ibs[k], sis[k])

    def wait_idx(k):
        pltpu.make_async_copy(
            xT_hbm.at[0, pl.ds(0, BB)], ibs[k], sis[k]).wait()

    def issue_gather(k):
        pltpu.async_copy(tok_hbm.at[ibs[k]], gbs[k], sgs[k])

    def wait_gather(k):
        pltpu.make_async_copy(
            tok_hbm.at[pl.ds(0, BB)], gbs[k], sgs[k]).wait()

    def issue_store(g, k):
        pltpu.async_copy(sbs[k], out_hbm.at[g, :, wid], sss[k])

    def wait_store(k):
        pltpu.make_async_copy(sbs[k], out_hbm.at[0, :, 0], sss[k]).wait()

    def transpose_add(g, k):
        gbuf = gbs[k]
        sbuf = sbs[k]
        pvs = [pos_v[g, pl.ds(16 * q, 16)] for q in range(4)]

        def s_body(i, carry):
            s0 = i * 2
            for ds in range(2):
                s = s0 + ds
                for q in range(4):
                    v = gbuf[s, pl.ds(16 * q, 16)] + pvs[q]
                    plsc.store_scatter(sbuf, [th_q[q], cb_q[q] + s], v)
            return carry

        lax.fori_loop(0, BB // 2, s_body, 0)

    issue_idx(0, 0)
    issue_idx(1, 1)
    wait_idx(0)
    issue_gather(0)

    def outer(t, carry):
        for b in range(NBUF):
            g = t * NBUF + b
            k1 = (b + 1) % NBUF
            k2 = (b + 2) % NBUF

            def stage1():
                wait_idx(k1)
                issue_gather(k1)

            if b == 3:
                @pl.when(t < NT - 1)
                def _():
                    stage1()
            else:
                stage1()

            def stage2():
                issue_idx(g + 2, k2)

            if b >= 2:
                @pl.when(t < NT - 1)
                def _():
                    stage2()
            else:
                stage2()

            wait_gather(b)

            @pl.when(t > 0)
            def _():
                wait_store(b)

            transpose_add(g, b)
            issue_store(g, b)
        return carry

    lax.fori_loop(0, NT, outer, 0)

    for b in range(NBUF):
        wait_store(b)


def kernel(x, token_table, pos_table):
    xT = jnp.transpose(x).astype(jnp.int32)
    o = _emb_kernel(xT, token_table, pos_table)
    o = o.reshape(L, HID // 8, NW, 8, 128)
    return o.transpose(2, 4, 0, 1, 3).reshape(B, L, HID)


# speedup vs baseline: 1.4848x; 1.3188x over previous
import functools

import jax
import jax.numpy as jnp
from jax import lax
from jax.experimental import pallas as pl
from jax.experimental.pallas import tpu as pltpu
from jax.experimental.pallas import tpu_sc as plsc

B = 4096
L = 200
HID = 64

NC = 2
NS = 16
NW = NC * NS
BB = B // NW
NBUF = 4
NT = L // NBUF


@functools.partial(
    pl.kernel,
    mesh=plsc.VectorSubcoreMesh(core_axis_name="c", subcore_axis_name="s"),
    compiler_params=pltpu.CompilerParams(
        use_tc_tiling_on_sc=False, needs_layout_passes=False),
    out_type=jax.ShapeDtypeStruct((L, HID // 8, NW, 8 * 128), jnp.float32),
    scratch_types=[
        pltpu.VMEM((L, HID), jnp.float32),
        pltpu.VMEM((BB,), jnp.int32),
        pltpu.VMEM((BB,), jnp.int32),
        pltpu.VMEM((BB,), jnp.int32),
        pltpu.VMEM((BB,), jnp.int32),
        pltpu.VMEM((BB, HID), jnp.float32),
        pltpu.VMEM((BB, HID), jnp.float32),
        pltpu.VMEM((BB, HID), jnp.float32),
        pltpu.VMEM((BB, HID), jnp.float32),
        pltpu.VMEM((HID // 8, 8 * 128), jnp.float32),
        pltpu.VMEM((HID // 8, 8 * 128), jnp.float32),
        pltpu.VMEM((HID // 8, 8 * 128), jnp.float32),
        pltpu.VMEM((HID // 8, 8 * 128), jnp.float32),
        pltpu.SemaphoreType.DMA,
        pltpu.SemaphoreType.DMA,
        pltpu.SemaphoreType.DMA,
        pltpu.SemaphoreType.DMA,
        pltpu.SemaphoreType.DMA,
        pltpu.SemaphoreType.DMA,
        pltpu.SemaphoreType.DMA,
        pltpu.SemaphoreType.DMA,
        pltpu.SemaphoreType.DMA,
        pltpu.SemaphoreType.DMA,
        pltpu.SemaphoreType.DMA,
        pltpu.SemaphoreType.DMA,
    ],
)
def _emb_kernel(xT_hbm, tok_hbm, pos_hbm, out_hbm,
                pos_v,
                ib0, ib1, ib2, ib3,
                gb0, gb1, gb2, gb3,
                sb0, sb1, sb2, sb3,
                si0, si1, si2, si3,
                sg0, sg1, sg2, sg3,
                ss0, ss1, ss2, ss3):
    ibs = (ib0, ib1, ib2, ib3)
    gbs = (gb0, gb1, gb2, gb3)
    sbs = (sb0, sb1, sb2, sb3)
    sis = (si0, si1, si2, si3)
    sgs = (sg0, sg1, sg2, sg3)
    sss = (ss0, ss1, ss2, ss3)

    wid = lax.axis_index("s") * NC + lax.axis_index("c")
    col0 = wid * BB

    pltpu.sync_copy(pos_hbm.at[pl.ds(0, L)], pos_v)

    iota = lax.iota(jnp.int32, 16)
    th_q = [(iota + 16 * q) >> 3 for q in range(4)]
    cb_q = [((iota + 16 * q) & 7) * 128 for q in range(4)]

    def issue_idx(g, k):
        pltpu.async_copy(xT_hbm.at[g, pl.ds(col0, BB)], ibs[k], sis[k])

    def wait_idx(k):
        pltpu.make_async_copy(
            xT_hbm.at[0, pl.ds(0, BB)], ibs[k], sis[k]).wait()

    def issue_gather(k):
        pltpu.async_copy(tok_hbm.at[ibs[k]], gbs[k], sgs[k])

    def wait_gather(k):
        pltpu.make_async_copy(
            tok_hbm.at[pl.ds(0, BB)], gbs[k], sgs[k]).wait()

    def issue_store(g, k):
        pltpu.async_copy(sbs[k], out_hbm.at[g, :, wid], sss[k])

    def wait_store(k):
        pltpu.make_async_copy(sbs[k], out_hbm.at[0, :, 0], sss[k]).wait()

    def transpose_add(g, k):
        gbuf = gbs[k]
        sbuf = sbs[k]
        pvs = [pos_v[g, pl.ds(16 * q, 16)] for q in range(4)]

        @plsc.parallel_loop(0, BB, step=1, unroll=8)
        def s_body(s):
            for q in range(4):
                v = gbuf[s, pl.ds(16 * q, 16)] + pvs[q]
                plsc.store_scatter(sbuf, [th_q[q], cb_q[q] + s], v)

    issue_idx(0, 0)
    issue_idx(1, 1)
    wait_idx(0)
    issue_gather(0)

    def outer(t, carry):
        for b in range(NBUF):
            g = t * NBUF + b
            k1 = (b + 1) % NBUF
            k2 = (b + 2) % NBUF

            def stage1():
                wait_idx(k1)
                issue_gather(k1)

            if b == 3:
                @pl.when(t < NT - 1)
                def _():
                    stage1()
            else:
                stage1()

            def stage2():
                issue_idx(g + 2, k2)

            if b >= 2:
                @pl.when(t < NT - 1)
                def _():
                    stage2()
            else:
                stage2()

            wait_gather(b)

            @pl.when(t > 0)
            def _():
                wait_store(b)

            transpose_add(g, b)
            issue_store(g, b)
        return carry

    lax.fori_loop(0, NT, outer, 0)

    for b in range(NBUF):
        wait_store(b)


def kernel(x, token_table, pos_table):
    xT = jnp.transpose(x).astype(jnp.int32)
    o = _emb_kernel(xT, token_table, pos_table)
    o = o.reshape(L, HID // 8, NW, 8, 128)
    return o.transpose(2, 4, 0, 1, 3).reshape(B, L, HID)
